# baseline (speedup 1.0000x reference)
import jax
import jax.numpy as jnp
from jax.experimental import pallas as pl
from jax.experimental.pallas import tpu as pltpu

B = 64
N = 576
C = 512
HID = 128
AG_HID = 102
KEEP = 115
NUM_KEEP = 288
BETA = 0.25

BB = 4


def _gelu(x):
    return 0.5 * x * (1.0 + jax.lax.erf(x * 0.7071067811865476))


def _mm_norm_col(s):
    smin = jnp.min(s, axis=0, keepdims=True)
    smax = jnp.max(s, axis=0, keepdims=True)
    return (s - smin) / (smax - smin + 1e-08)


def _body(tok_ref, glb_ref, jlt_ref, spw1_ref, spb1_ref, spw2_ref, spb2_ref,
          lng_ref, lnb_ref, agw1_ref, agb1_ref, agw2_ref, agb2_ref,
          scale_ref, out_ref):
    for b in range(BB):
        _sample(b, tok_ref, glb_ref, jlt_ref, spw1_ref, spb1_ref, spw2_ref,
                spb2_ref, lng_ref, lnb_ref, agw1_ref, agb1_ref, agw2_ref,
                agb2_ref, scale_ref, out_ref)


def _sample(b, tok_ref, glb_ref, jlt_ref, spw1_ref, spb1_ref, spw2_ref,
            spb2_ref, lng_ref, lnb_ref, agw1_ref, agb1_ref, agw2_ref,
            agb2_ref, scale_ref, out_ref):
    m = pl.program_id(0)
    bf = jnp.bfloat16
    toks = tok_ref[0, b]
    glbs = glb_ref[b]
    toks_bf = toks.astype(bf)

    sumsq = jnp.sum(toks * toks, axis=-1, keepdims=True)
    tnorm = jnp.maximum(jnp.sqrt(sumsq), 1e-12)
    gnorm = jnp.maximum(
        jnp.sqrt(jnp.sum(glbs * glbs, axis=-1, keepdims=True)), 1e-12)
    pn = toks / tnorm
    gn = glbs / gnorm
    c0 = jnp.sum(pn * gn[0:1], axis=-1, keepdims=True)
    c1 = jnp.sum(pn * gn[1:2], axis=-1, keepdims=True)
    c2 = jnp.sum(pn * gn[2:3], axis=-1, keepdims=True)
    m0 = _mm_norm_col(c0)
    m1 = _mm_norm_col(c1)
    m2 = _mm_norm_col(c2)
    s_im = jnp.where(m == 0, m0, jnp.where(m == 1, m1, m2))
    s_m2 = jnp.where(m == 0, m1, m0)
    s_m3 = jnp.where(m == 2, m1, m2)

    h = jnp.dot(toks_bf, spw1_ref[0],
                preferred_element_type=jnp.float32) + spb1_ref[0]
    h = _gelu(h)
    sp = jnp.dot(h.astype(bf), spw2_ref[0],
                 preferred_element_type=jnp.float32) + spb2_ref[0, 0, 0]
    s_pred = jax.nn.sigmoid(sp)

    s_col = (1.0 - 2.0 * BETA) * s_pred \
        + BETA * (s_m2 + s_m3 + 2.0 * s_im)
    s_row = jnp.transpose(s_col)

    si_col = jax.lax.bitcast_convert_type(s_col, jnp.int32)
    si_row = jax.lax.bitcast_convert_type(s_row, jnp.int32)
    thresh = si_col - jlt_ref[...]
    beats = jnp.where(si_row > thresh, 1.0, 0.0).astype(bf)
    rank = jnp.dot(beats, jnp.ones((N, 1), bf),
                   preferred_element_type=jnp.float32)
    keep_row = jnp.transpose(rank) < NUM_KEEP

    s_nk = jnp.where(keep_row, jnp.float32(-1e30), s_row)
    e = jnp.exp(s_nk - jnp.max(s_nk, axis=-1, keepdims=True))
    e = jnp.where(keep_row, 0.0, e)
    wts = e / jnp.sum(e, axis=-1, keepdims=True)
    extra = jnp.dot(wts.astype(bf), toks_bf,
                    preferred_element_type=jnp.float32)

    mu = jnp.dot(toks_bf, jnp.ones((C, 1), bf),
                 preferred_element_type=jnp.float32) * (1.0 / C)
    var = sumsq * (1.0 / C) - mu * mu
    inv = jax.lax.rsqrt(var + 1e-05)
    xn = ((toks - mu) * inv * lng_ref[0] + lnb_ref[0]).astype(bf)
    h1 = jnp.dot(xn, agw1_ref[0],
                 preferred_element_type=jnp.float32) + agb1_ref[0]
    h1 = _gelu(h1)
    w = jnp.dot(h1.astype(bf), agw2_ref[0],
                preferred_element_type=jnp.float32) + agb2_ref[0]
    w = jnp.transpose(w) * scale_ref[0, 0, 0]
    w = jnp.where(keep_row, w, w - 1e10)
    w = w - jnp.max(w, axis=-1, keepdims=True)
    ew = jnp.exp(w)
    w = ew / jnp.sum(ew, axis=-1, keepdims=True)
    aggr = jnp.dot(w.astype(bf), toks_bf,
                   preferred_element_type=jnp.float32)

    out_ref[0, b, :KEEP, :] = aggr
    out_ref[0, b, KEEP:, :] = extra


def kernel(rgb, nir, tir, rgb_global, nir_global, tir_global,
           sp_w1, sp_b1, sp_w2, sp_b2, ag_ln_g, ag_ln_b,
           ag_w1, ag_b1, ag_w2, ag_b2, ag_scale):
    bf = jnp.bfloat16
    toks = jnp.stack([rgb, nir, tir])
    glbs = jnp.stack([rgb_global, nir_global, tir_global],
                     axis=1)
    jlt = (jnp.arange(N, dtype=jnp.int32)[None, :]
           < jnp.arange(N, dtype=jnp.int32)[:, None]).astype(jnp.int32)

    grid = (3, B // BB)
    out = pl.pallas_call(
        _body,
        grid=grid,
        in_specs=[
            pl.BlockSpec((1, BB, N, C), lambda m, i: (m, i, 0, 0)),
            pl.BlockSpec((BB, 3, C), lambda m, i: (i, 0, 0)),
            pl.BlockSpec((N, N), lambda m, i: (0, 0)),
            pl.BlockSpec((1, C, HID), lambda m, i: (m, 0, 0)),
            pl.BlockSpec((1, 1, HID), lambda m, i: (m, 0, 0)),
            pl.BlockSpec((1, HID, 1), lambda m, i: (m, 0, 0)),
            pl.BlockSpec((1, 1, 1), lambda m, i: (m, 0, 0)),
            pl.BlockSpec((1, 1, C), lambda m, i: (m, 0, 0)),
            pl.BlockSpec((1, 1, C), lambda m, i: (m, 0, 0)),
            pl.BlockSpec((1, C, AG_HID), lambda m, i: (m, 0, 0)),
            pl.BlockSpec((1, 1, AG_HID), lambda m, i: (m, 0, 0)),
            pl.BlockSpec((1, AG_HID, KEEP), lambda m, i: (m, 0, 0)),
            pl.BlockSpec((1, 1, KEEP), lambda m, i: (m, 0, 0)),
            pl.BlockSpec((1, 1, 1), lambda m, i: (m, 0, 0)),
        ],
        out_specs=pl.BlockSpec((1, BB, KEEP + 1, C), lambda m, i: (m, i, 0, 0)),
        out_shape=jax.ShapeDtypeStruct((3, B, KEEP + 1, C), jnp.float32),
        compiler_params=pltpu.CompilerParams(
            dimension_semantics=("parallel", "parallel"),
        ),
    )(toks, glbs, jlt, sp_w1.astype(bf), sp_b1.reshape(3, 1, HID),
      sp_w2.astype(bf), sp_b2.reshape(3, 1, 1),
      ag_ln_g.reshape(3, 1, C), ag_ln_b.reshape(3, 1, C),
      ag_w1.astype(bf), ag_b1.reshape(3, 1, AG_HID), ag_w2.astype(bf),
      ag_b2.reshape(3, 1, KEEP), ag_scale.reshape(3, 1, 1))
    return (out[0], out[1], out[2])


# speedup vs baseline: 1.3091x; 1.3091x over previous
import jax
import jax.numpy as jnp
from jax.experimental import pallas as pl
from jax.experimental.pallas import tpu as pltpu

B = 64
N = 576
C = 512
HID = 128
AG_HID = 102
KEEP = 115
NUM_KEEP = 288
BETA = 0.25


def _gelu(x):
    return 0.5 * x * (1.0 + jax.lax.erf(x * 0.7071067811865476))


def _mm_norm_col(s):
    smin = jnp.min(s, axis=0, keepdims=True)
    smax = jnp.max(s, axis=0, keepdims=True)
    return (s - smin) / (smax - smin + 1e-08)


BB = 4


def _body(tok_ref, glb_ref, spw1_ref, spb1_ref, spw2_ref, spb2_ref,
          lng_ref, lnb_ref, agw1_ref, agb1_ref, agw2_ref, agb2_ref,
          scale_ref, out_ref):
    for b in range(BB):
        _sample(b, tok_ref, glb_ref, spw1_ref, spb1_ref, spw2_ref, spb2_ref,
                lng_ref, lnb_ref, agw1_ref, agb1_ref, agw2_ref, agb2_ref,
                scale_ref, out_ref)


def _sample(b, tok_ref, glb_ref, spw1_ref, spb1_ref, spw2_ref, spb2_ref,
            lng_ref, lnb_ref, agw1_ref, agb1_ref, agw2_ref, agb2_ref,
            scale_ref, out_ref):
    m = pl.program_id(0)
    toks = tok_ref[0, b]
    glbs = glb_ref[b]

    sumsq = jnp.sum(toks * toks, axis=-1, keepdims=True)
    tnorm = jnp.maximum(jnp.sqrt(sumsq), 1e-12)
    gnorm = jnp.maximum(
        jnp.sqrt(jnp.sum(glbs * glbs, axis=-1, keepdims=True)), 1e-12)
    pn = toks / tnorm
    gn = glbs / gnorm
    c0 = jnp.sum(pn * gn[0:1], axis=-1, keepdims=True)
    c1 = jnp.sum(pn * gn[1:2], axis=-1, keepdims=True)
    c2 = jnp.sum(pn * gn[2:3], axis=-1, keepdims=True)
    m0 = _mm_norm_col(c0)
    m1 = _mm_norm_col(c1)
    m2 = _mm_norm_col(c2)
    s_im = jnp.where(m == 0, m0, jnp.where(m == 1, m1, m2))
    s_m2 = jnp.where(m == 0, m1, m0)
    s_m3 = jnp.where(m == 2, m1, m2)

    bf = jnp.bfloat16
    h = jnp.dot(toks.astype(bf), spw1_ref[0].astype(bf),
                preferred_element_type=jnp.float32) + spb1_ref[0]
    h = _gelu(h)
    sp = jnp.dot(h.astype(bf), spw2_ref[0].astype(bf),
                 preferred_element_type=jnp.float32) + spb2_ref[0, 0, 0]
    s_pred = jax.nn.sigmoid(sp)

    s_col = (1.0 - 2.0 * BETA) * s_pred \
        + BETA * (s_m2 + s_m3 + 2.0 * s_im)
    s_row = jnp.transpose(s_col)

    jlt = jax.lax.broadcasted_iota(jnp.int32, (N, N), 1) < \
        jax.lax.broadcasted_iota(jnp.int32, (N, N), 0)
    beats = jnp.where((s_row > s_col) | ((s_row == s_col) & jlt), 1.0, 0.0)
    rank = jnp.sum(beats, axis=1, keepdims=True)
    keep = rank < NUM_KEEP
    keep_row = jnp.transpose(keep)

    s_nk = jnp.where(keep, jnp.float32(-1e30), s_col)
    e = jnp.exp(s_nk - jnp.max(s_nk, axis=0, keepdims=True))
    e = jnp.where(keep, 0.0, e)
    wts = e / jnp.sum(e, axis=0, keepdims=True)
    extra = jnp.sum(toks * wts, axis=0, keepdims=True)

    mu = jnp.mean(toks, axis=-1, keepdims=True)
    var = sumsq * (1.0 / C) - mu * mu
    inv = jax.lax.rsqrt(var + 1e-05)
    xn = (toks - mu) * inv * lng_ref[0] + lnb_ref[0]
    h1 = jnp.dot(xn, agw1_ref[0], preferred_element_type=jnp.float32) \
        + agb1_ref[0]
    h1 = _gelu(h1)
    w = jnp.dot(h1, agw2_ref[0], preferred_element_type=jnp.float32) \
        + agb2_ref[0]
    w = jnp.transpose(w) * scale_ref[0, 0, 0]
    w = jnp.where(keep_row, w, w - 1e10)
    w = w - jnp.max(w, axis=-1, keepdims=True)
    ew = jnp.exp(w)
    w = ew / jnp.sum(ew, axis=-1, keepdims=True)
    aggr = jnp.dot(w, toks, preferred_element_type=jnp.float32)

    out_ref[0, b, :KEEP, :] = aggr
    out_ref[0, b, KEEP:, :] = extra


def kernel(rgb, nir, tir, rgb_global, nir_global, tir_global,
           sp_w1, sp_b1, sp_w2, sp_b2, ag_ln_g, ag_ln_b,
           ag_w1, ag_b1, ag_w2, ag_b2, ag_scale):
    toks = jnp.stack([rgb, nir, tir])
    glbs = jnp.stack([rgb_global, nir_global, tir_global],
                     axis=1)

    grid = (3, B // BB)
    out = pl.pallas_call(
        _body,
        grid=grid,
        in_specs=[
            pl.BlockSpec((1, BB, N, C), lambda m, i: (m, i, 0, 0)),
            pl.BlockSpec((BB, 3, C), lambda m, i: (i, 0, 0)),
            pl.BlockSpec((1, C, HID), lambda m, i: (m, 0, 0)),
            pl.BlockSpec((1, 1, HID), lambda m, i: (m, 0, 0)),
            pl.BlockSpec((1, HID, 1), lambda m, i: (m, 0, 0)),
            pl.BlockSpec((1, 1, 1), lambda m, i: (m, 0, 0)),
            pl.BlockSpec((1, 1, C), lambda m, i: (m, 0, 0)),
            pl.BlockSpec((1, 1, C), lambda m, i: (m, 0, 0)),
            pl.BlockSpec((1, C, AG_HID), lambda m, i: (m, 0, 0)),
            pl.BlockSpec((1, 1, AG_HID), lambda m, i: (m, 0, 0)),
            pl.BlockSpec((1, AG_HID, KEEP), lambda m, i: (m, 0, 0)),
            pl.BlockSpec((1, 1, KEEP), lambda m, i: (m, 0, 0)),
            pl.BlockSpec((1, 1, 1), lambda m, i: (m, 0, 0)),
        ],
        out_specs=pl.BlockSpec((1, BB, KEEP + 1, C), lambda m, i: (m, i, 0, 0)),
        out_shape=jax.ShapeDtypeStruct((3, B, KEEP + 1, C), jnp.float32),
        compiler_params=pltpu.CompilerParams(
            dimension_semantics=("parallel", "parallel"),
        ),
    )(toks, glbs, sp_w1, sp_b1.reshape(3, 1, HID), sp_w2,
      sp_b2.reshape(3, 1, 1), ag_ln_g.reshape(3, 1, C), ag_ln_b.reshape(3, 1, C),
      ag_w1, ag_b1.reshape(3, 1, AG_HID), ag_w2, ag_b2.reshape(3, 1, KEEP),
      ag_scale.reshape(3, 1, 1))
    return (out[0], out[1], out[2])
